# baseline (speedup 1.0000x reference)
import functools

import jax
import jax.numpy as jnp
from jax import lax
from jax.experimental import pallas as pl
from jax.experimental.pallas import tpu as pltpu
from jax.experimental.pallas import tpu_sc as plsc

N_ROWS = 50000
K = 33
KN = K - 1
NW = 32
C = 128
CHUNKS = 391
LASTC = CHUNKS - 1
TAIL = N_ROWS - LASTC * C
KMAX = -(-CHUNKS // NW)
NPAD1 = 50176


def _feat_body(tri_ref, bary_ref, out_ref):
    t = tri_ref[...]

    def edge(a, b):
        dx = t[3 * a + 0] - t[3 * b + 0]
        dy = t[3 * a + 1] - t[3 * b + 1]
        dz = t[3 * a + 2] - t[3 * b + 2]
        return jnp.sqrt(dx * dx + dy * dy + dz * dz)

    e01 = edge(0, 1)
    e02 = edge(0, 2)
    e12 = edge(1, 2)
    out_ref[0] = jnp.minimum(jnp.minimum(e01, e02), e12)
    out_ref[1] = jnp.maximum(jnp.maximum(e01, e02), e12)
    b = bary_ref[...]
    out_ref[2] = b[0]
    out_ref[3] = b[1]
    out_ref[4] = b[2]
    z = jnp.zeros_like(e01)
    out_ref[5] = z
    out_ref[6] = z
    out_ref[7] = z


def _features(tri_t, bary_t):
    bs = 56
    nblk = NPAD1 // (bs * 128)
    return pl.pallas_call(
        _feat_body,
        grid=(nblk,),
        in_specs=[
            pl.BlockSpec((9, bs, 128), lambda i: (0, i, 0)),
            pl.BlockSpec((3, bs, 128), lambda i: (0, i, 0)),
        ],
        out_specs=pl.BlockSpec((8, bs, 128), lambda i: (0, i, 0)),
        out_shape=jax.ShapeDtypeStruct((8, NPAD1 // 128, 128), jnp.float32),
    )(tri_t, bary_t)


def _gather_body(f8_hbm, idxt_hbm, out_hbm, idx_v, rows_v, out_v,
                 gsem0, gsem1, osem0, osem1):
    wid = lax.axis_index("s") * 2 + lax.axis_index("c")
    iota = lax.iota(jnp.int32, 16)
    gsems = (gsem0, gsem1)
    osems = (osem0, osem1)

    def load_idx(g, b):
        @pl.when(g < LASTC)
        def _():
            pltpu.sync_copy(idxt_hbm.at[:, pl.ds(g * C, C)], idx_v.at[b])

        @pl.when(g == LASTC)
        def _():
            pltpu.sync_copy(idxt_hbm.at[:, pl.ds(LASTC * C, TAIL)],
                            idx_v.at[b].at[:, pl.ds(0, TAIL)])

    def fire_gathers(b, sem):
        for j in range(K):
            pltpu.async_copy(f8_hbm.at[idx_v.at[b, j]], rows_v.at[b, j], sem)

    def drain_gathers(b, sem):
        for j in range(K):
            pltpu.make_async_copy(f8_hbm.at[idx_v.at[b, j]],
                                  rows_v.at[b, j], sem).wait()

    load_idx(wid, 0)
    fire_gathers(0, gsems[0])

    def outer_body(kk, carry):
        for b in range(2):
            k = kk * 2 + b
            g = wid + k * NW

            @pl.when(g < CHUNKS)
            def _(b=b, k=k, g=g):
                gn = g + NW

                @pl.when(gn < CHUNKS)
                def _():
                    load_idx(gn, 1 - b)
                    fire_gathers(1 - b, gsems[1 - b])

                drain_gathers(b, gsems[b])

                @pl.when(kk >= 1)
                def _():
                    pltpu.make_async_copy(out_v.at[b], out_hbm.at[:, :, g],
                                          osems[b]).wait()

                rv = rows_v.at[b]
                zsp = jnp.full((16,), 0, jnp.int32)
                vvecs = [iota + (v * 16) for v in range(8)]
                for c in range(5):
                    csp = jnp.full((16,), c, jnp.int32)
                    cvals = [plsc.load_gather(rv, [zsp, vvecs[v], csp])
                             for v in range(8)]

                    def j_body(jh, jcarry, c=c, csp=csp, cvals=cvals,
                               rv=rv, b=b):
                        for dj in range(2):
                            j = jh * 2 + dj
                            jt = j // 8
                            js = j - jt * 8
                            jsp = jnp.full((16,), j + 1, jnp.int32)
                            for v in range(8):
                                nval = plsc.load_gather(
                                    rv, [jsp, vvecs[v], csp])
                                out_v[b, c, jt, js, pl.ds(v * 16, 16)] = (
                                    cvals[v] - nval)
                        return jcarry

                    lax.fori_loop(0, KN // 2, j_body, 0)

                pltpu.async_copy(out_v.at[b], out_hbm.at[:, :, g], osems[b])

        return carry

    lax.fori_loop(0, (KMAX + 1) // 2, outer_body, 0)

    for b in range(2):
        pltpu.make_async_copy(out_v.at[b], out_hbm.at[:, :, 0],
                              osems[b]).wait()


def _gather(f8, idxt):
    mesh = plsc.VectorSubcoreMesh(core_axis_name="c", subcore_axis_name="s",
                                  num_cores=2, num_subcores=16)
    run = functools.partial(
        pl.kernel,
        out_type=jax.ShapeDtypeStruct((5, 4, CHUNKS, 8, C), jnp.float32),
        mesh=mesh,
        compiler_params=pltpu.CompilerParams(use_tc_tiling_on_sc=False,
                                             needs_layout_passes=False),
        scratch_types=[
            pltpu.VMEM((2, K, C), jnp.int32),
            pltpu.VMEM((2, K, C, 8), jnp.float32),
            pltpu.VMEM((2, 5, 4, 8, C), jnp.float32),
            pltpu.SemaphoreType.DMA,
            pltpu.SemaphoreType.DMA,
            pltpu.SemaphoreType.DMA,
            pltpu.SemaphoreType.DMA,
        ],
    )(_gather_body)
    return run(f8, idxt)


def kernel(triangles, barycenters, indices_neigh_tri, number_neigh_tri):
    n = triangles.shape[0]
    pad = NPAD1 - n
    tri_t = jnp.pad(triangles.reshape(n, 9), ((0, pad), (0, 0))).T
    tri_t = tri_t.reshape(9, NPAD1 // 128, 128)
    bary_t = jnp.pad(barycenters, ((0, pad), (0, 0))).T
    bary_t = bary_t.reshape(3, NPAD1 // 128, 128)
    f8t = _features(tri_t, bary_t)
    f8 = f8t.transpose(1, 2, 0).reshape(NPAD1, 8)
    idxt = indices_neigh_tri.astype(jnp.int32).T
    out5 = _gather(f8, idxt)
    out = jnp.transpose(out5, (2, 4, 1, 3, 0)).reshape(CHUNKS * C, KN, 5)
    return out[:n]


# speedup vs baseline: 1.0047x; 1.0047x over previous
import functools

import jax
import jax.numpy as jnp
from jax import lax
from jax.experimental import pallas as pl
from jax.experimental.pallas import tpu as pltpu
from jax.experimental.pallas import tpu_sc as plsc

N_ROWS = 50000
K = 33
KN = K - 1
NW = 32
C = 128
CHUNKS = 391
LASTC = CHUNKS - 1
TAIL = N_ROWS - LASTC * C
KMAX = -(-CHUNKS // NW)
NPAD1 = 50176


def _feat_body(tri_ref, bary_ref, out_ref):
    t = tri_ref[...]

    def edge(a, b):
        dx = t[3 * a + 0] - t[3 * b + 0]
        dy = t[3 * a + 1] - t[3 * b + 1]
        dz = t[3 * a + 2] - t[3 * b + 2]
        return jnp.sqrt(dx * dx + dy * dy + dz * dz)

    e01 = edge(0, 1)
    e02 = edge(0, 2)
    e12 = edge(1, 2)
    out_ref[0] = jnp.minimum(jnp.minimum(e01, e02), e12)
    out_ref[1] = jnp.maximum(jnp.maximum(e01, e02), e12)
    b = bary_ref[...]
    out_ref[2] = b[0]
    out_ref[3] = b[1]
    out_ref[4] = b[2]
    z = jnp.zeros_like(e01)
    out_ref[5] = z
    out_ref[6] = z
    out_ref[7] = z


def _features(tri_t, bary_t):
    bs = 56
    nblk = NPAD1 // (bs * 128)
    return pl.pallas_call(
        _feat_body,
        grid=(nblk,),
        in_specs=[
            pl.BlockSpec((9, bs, 128), lambda i: (0, i, 0)),
            pl.BlockSpec((3, bs, 128), lambda i: (0, i, 0)),
        ],
        out_specs=pl.BlockSpec((8, bs, 128), lambda i: (0, i, 0)),
        out_shape=jax.ShapeDtypeStruct((8, NPAD1 // 128, 128), jnp.float32),
    )(tri_t, bary_t)


def _gather_body(f8_hbm, idxt_hbm, out_hbm, idx_v, rows_v, out_v,
                 gsem0, gsem1, osem0, osem1):
    wid = lax.axis_index("s") * 2 + lax.axis_index("c")
    iota = lax.iota(jnp.int32, 16)
    gsems = (gsem0, gsem1)
    osems = (osem0, osem1)

    def load_idx(g, b):
        @pl.when(g < LASTC)
        def _():
            pltpu.sync_copy(idxt_hbm.at[:, pl.ds(g * C, C)], idx_v.at[b])

        @pl.when(g == LASTC)
        def _():
            pltpu.sync_copy(idxt_hbm.at[:, pl.ds(LASTC * C, TAIL)],
                            idx_v.at[b].at[:, pl.ds(0, TAIL)])

    def fire_gathers(b, sem):
        for j in range(K):
            pltpu.async_copy(f8_hbm.at[idx_v.at[b, j]], rows_v.at[b, j], sem)

    def drain_gathers(b, sem):
        for j in range(K):
            pltpu.make_async_copy(f8_hbm.at[idx_v.at[b, j]],
                                  rows_v.at[b, j], sem).wait()

    load_idx(wid, 0)
    fire_gathers(0, gsems[0])

    def outer_body(kk, carry):
        for b in range(2):
            k = kk * 2 + b
            g = wid + k * NW

            @pl.when(g < CHUNKS)
            def _(b=b, k=k, g=g):
                gn = g + NW

                @pl.when(gn < CHUNKS)
                def _():
                    load_idx(gn, 1 - b)
                    fire_gathers(1 - b, gsems[1 - b])

                drain_gathers(b, gsems[b])

                @pl.when(kk >= 1)
                def _():
                    pltpu.make_async_copy(out_v.at[b], out_hbm.at[:, :, g],
                                          osems[b]).wait()

                rv = rows_v.at[b]
                zsp = jnp.full((16,), 0, jnp.int32)
                vvecs = [iota + (v * 16) for v in range(8)]
                for c in range(5):
                    csp = jnp.full((16,), c, jnp.int32)
                    cvals = [plsc.load_gather(rv, [zsp, vvecs[v], csp])
                             for v in range(8)]

                    def j_body(j, jcarry, c=c, csp=csp, cvals=cvals,
                               rv=rv, b=b):
                        jt = j // 8
                        js = j - jt * 8
                        jsp = jnp.full((16,), j + 1, jnp.int32)
                        for v in range(8):
                            nval = plsc.load_gather(rv, [jsp, vvecs[v], csp])
                            out_v[b, c, jt, js, pl.ds(v * 16, 16)] = (
                                cvals[v] - nval)
                        return jcarry

                    lax.fori_loop(0, KN, j_body, 0)

                pltpu.async_copy(out_v.at[b], out_hbm.at[:, :, g], osems[b])

        return carry

    lax.fori_loop(0, (KMAX + 1) // 2, outer_body, 0)

    for b in range(2):
        pltpu.make_async_copy(out_v.at[b], out_hbm.at[:, :, 0],
                              osems[b]).wait()


def _gather(f8, idxt):
    mesh = plsc.VectorSubcoreMesh(core_axis_name="c", subcore_axis_name="s",
                                  num_cores=2, num_subcores=16)
    run = functools.partial(
        pl.kernel,
        out_type=jax.ShapeDtypeStruct((5, 4, CHUNKS, 8, C), jnp.float32),
        mesh=mesh,
        compiler_params=pltpu.CompilerParams(use_tc_tiling_on_sc=False,
                                             needs_layout_passes=False),
        scratch_types=[
            pltpu.VMEM((2, K, C), jnp.int32),
            pltpu.VMEM((2, K, C, 8), jnp.float32),
            pltpu.VMEM((2, 5, 4, 8, C), jnp.float32),
            pltpu.SemaphoreType.DMA,
            pltpu.SemaphoreType.DMA,
            pltpu.SemaphoreType.DMA,
            pltpu.SemaphoreType.DMA,
        ],
    )(_gather_body)
    return run(f8, idxt)


def kernel(triangles, barycenters, indices_neigh_tri, number_neigh_tri):
    n = triangles.shape[0]
    pad = NPAD1 - n
    tri_t = jnp.pad(triangles.reshape(n, 9), ((0, pad), (0, 0))).T
    tri_t = tri_t.reshape(9, NPAD1 // 128, 128)
    bary_t = jnp.pad(barycenters, ((0, pad), (0, 0))).T
    bary_t = bary_t.reshape(3, NPAD1 // 128, 128)
    f8t = _features(tri_t, bary_t)
    f8 = f8t.transpose(1, 2, 0).reshape(NPAD1, 8)
    idxt = indices_neigh_tri.astype(jnp.int32).T
    out5 = _gather(f8, idxt)
    out = jnp.transpose(out5, (2, 4, 1, 3, 0)).reshape(CHUNKS * C, KN, 5)
    return out[:n]


# speedup vs baseline: 1.0201x; 1.0153x over previous
import functools

import jax
import jax.numpy as jnp
from jax import lax
from jax.experimental import pallas as pl
from jax.experimental.pallas import tpu as pltpu
from jax.experimental.pallas import tpu_sc as plsc

N_ROWS = 50000
K = 33
KN = K - 1
NW = 32
C = 128
CHUNKS = 391
LASTC = CHUNKS - 1
TAIL = N_ROWS - LASTC * C
KMAX = -(-CHUNKS // NW)
NPAD1 = 50176


def _feat_body(tri_ref, bary_ref, out_ref):
    t = tri_ref[...]

    def edge(a, b):
        dx = t[3 * a + 0] - t[3 * b + 0]
        dy = t[3 * a + 1] - t[3 * b + 1]
        dz = t[3 * a + 2] - t[3 * b + 2]
        return jnp.sqrt(dx * dx + dy * dy + dz * dz)

    e01 = edge(0, 1)
    e02 = edge(0, 2)
    e12 = edge(1, 2)
    out_ref[0] = jnp.minimum(jnp.minimum(e01, e02), e12)
    out_ref[1] = jnp.maximum(jnp.maximum(e01, e02), e12)
    b = bary_ref[...]
    out_ref[2] = b[0]
    out_ref[3] = b[1]
    out_ref[4] = b[2]
    z = jnp.zeros_like(e01)
    out_ref[5] = z
    out_ref[6] = z
    out_ref[7] = z


def _features(tri_t, bary_t):
    bs = 56
    nblk = NPAD1 // (bs * 128)
    return pl.pallas_call(
        _feat_body,
        grid=(nblk,),
        in_specs=[
            pl.BlockSpec((9, bs, 128), lambda i: (0, i, 0)),
            pl.BlockSpec((3, bs, 128), lambda i: (0, i, 0)),
        ],
        out_specs=pl.BlockSpec((8, bs, 128), lambda i: (0, i, 0)),
        out_shape=jax.ShapeDtypeStruct((8, NPAD1 // 128, 128), jnp.float32),
    )(tri_t, bary_t)


def _gather_body(f8_hbm, idxt_hbm, out_hbm, idx_v, rows_v, out_v, tab_sp,
                 gsem0, gsem1, osem):
    wid = lax.axis_index("s") * 2 + lax.axis_index("c")
    iota = lax.iota(jnp.int32, 16)
    gsems = (gsem0, gsem1)

    sid = lax.axis_index("s")
    rows_per_tile = NPAD1 // 16
    pltpu.sync_copy(f8_hbm.at[pl.ds(sid * rows_per_tile, rows_per_tile)],
                    tab_sp.at[pl.ds(sid * rows_per_tile, rows_per_tile)])
    plsc.subcore_barrier()

    def load_idx(g, b):
        @pl.when(g < LASTC)
        def _():
            pltpu.sync_copy(idxt_hbm.at[:, pl.ds(g * C, C)], idx_v.at[b])

        @pl.when(g == LASTC)
        def _():
            pltpu.sync_copy(idxt_hbm.at[:, pl.ds(LASTC * C, TAIL)],
                            idx_v.at[b].at[:, pl.ds(0, TAIL)])

    def fire_gathers(b, sem):
        for j in range(K):
            pltpu.async_copy(tab_sp.at[idx_v.at[b, j]], rows_v.at[b, j], sem)

    def drain_gathers(b, sem):
        for j in range(K):
            pltpu.make_async_copy(tab_sp.at[idx_v.at[b, j]],
                                  rows_v.at[b, j], sem).wait()

    load_idx(wid, 0)
    fire_gathers(0, gsems[0])

    def outer_body(kk, carry):
        for b in range(2):
            k = kk * 2 + b
            g = wid + k * NW

            @pl.when(g < CHUNKS)
            def _(b=b, k=k, g=g):
                gn = g + NW

                @pl.when(gn < CHUNKS)
                def _():
                    load_idx(gn, 1 - b)
                    fire_gathers(1 - b, gsems[1 - b])

                drain_gathers(b, gsems[b])

                @pl.when(k >= 1)
                def _():
                    pltpu.make_async_copy(out_v, out_hbm.at[:, :, g],
                                          osem).wait()

                rv = rows_v.at[b]
                zsp = jnp.full((16,), 0, jnp.int32)
                vvecs = [iota + (v * 16) for v in range(8)]
                for c in range(5):
                    csp = jnp.full((16,), c, jnp.int32)
                    cvals = [plsc.load_gather(rv, [zsp, vvecs[v], csp])
                             for v in range(8)]

                    def j_body(j, jcarry, c=c, csp=csp, cvals=cvals,
                               rv=rv, b=b):
                        jt = j // 8
                        js = j - jt * 8
                        jsp = jnp.full((16,), j + 1, jnp.int32)
                        for v in range(8):
                            nval = plsc.load_gather(rv, [jsp, vvecs[v], csp])
                            out_v[c, jt, js, pl.ds(v * 16, 16)] = (
                                cvals[v] - nval)
                        return jcarry

                    lax.fori_loop(0, KN, j_body, 0)

                pltpu.async_copy(out_v, out_hbm.at[:, :, g], osem)

        return carry

    lax.fori_loop(0, (KMAX + 1) // 2, outer_body, 0)

    pltpu.make_async_copy(out_v, out_hbm.at[:, :, 0], osem).wait()


def _gather(f8, idxt):
    mesh = plsc.VectorSubcoreMesh(core_axis_name="c", subcore_axis_name="s",
                                  num_cores=2, num_subcores=16)
    run = functools.partial(
        pl.kernel,
        out_type=jax.ShapeDtypeStruct((5, 4, CHUNKS, 8, C), jnp.float32),
        mesh=mesh,
        compiler_params=pltpu.CompilerParams(use_tc_tiling_on_sc=False,
                                             needs_layout_passes=False),
        scratch_types=[
            pltpu.VMEM((2, K, C), jnp.int32),
            pltpu.VMEM((2, K, C, 8), jnp.float32),
            pltpu.VMEM((5, 4, 8, C), jnp.float32),
            pltpu.VMEM_SHARED((NPAD1, 8), jnp.float32),
            pltpu.SemaphoreType.DMA,
            pltpu.SemaphoreType.DMA,
            pltpu.SemaphoreType.DMA,
        ],
    )(_gather_body)
    return run(f8, idxt)


def kernel(triangles, barycenters, indices_neigh_tri, number_neigh_tri):
    n = triangles.shape[0]
    pad = NPAD1 - n
    tri_t = jnp.pad(triangles.reshape(n, 9), ((0, pad), (0, 0))).T
    tri_t = tri_t.reshape(9, NPAD1 // 128, 128)
    bary_t = jnp.pad(barycenters, ((0, pad), (0, 0))).T
    bary_t = bary_t.reshape(3, NPAD1 // 128, 128)
    f8t = _features(tri_t, bary_t)
    f8 = f8t.transpose(1, 2, 0).reshape(NPAD1, 8)
    idxt = indices_neigh_tri.astype(jnp.int32).T
    out5 = _gather(f8, idxt)
    out = jnp.transpose(out5, (2, 4, 1, 3, 0)).reshape(CHUNKS * C, KN, 5)
    return out[:n]
